# baseline (speedup 1.0000x reference)
import functools

import jax
import jax.numpy as jnp
from jax import lax
from jax.experimental import pallas as pl
from jax.experimental.pallas import tpu as pltpu
from jax.experimental.pallas import tpu_sc as plsc

VOCAB = 1000000
EMB = 64
SEQ = 200
BATCH = 4096

NC = 2
NS = 16
NW = NC * NS
B_PER_W = BATCH // NW

CHUNK0 = 128
CHUNK1 = SEQ - CHUNK0

RW = 512
NSUB = 8
NFULL = VOCAB // RW
NMAIN = 1952
PER_TILE = NMAIN // NW
VTAIL = VOCAB - NFULL * RW


def _repack_chunk(tt_v, out_p, buf, rowvs, zero16, half):
    def body(c, _):
        colv = zero16 + c
        vals = [tt_v[buf, c, pl.ds(256 * half + 16 * g, 16)]
                for g in range(16)]
        for g in range(16):
            plsc.store_scatter(out_p, [rowvs[g], colv], vals[g])
        return 0

    lax.fori_loop(0, EMB, body, 0, unroll=1)


def _compact_half(out_p, out_v):
    def body(r, _):
        s0 = 2 * r
        for k in range(4):
            out_v[r, pl.ds(16 * k, 16)] = out_p[s0, pl.ds(16 * k, 16)]
        for k in range(4):
            out_v[r, pl.ds(64 + 16 * k, 16)] = out_p[s0 + 1, pl.ds(16 * k, 16)]
        return 0

    lax.fori_loop(0, RW // 4, body, 0, unroll=2)


def _repack_body(tt_hbm, tail_hbm, out_hbm, tt_v, out_p, out_v, isem, osem):
    wid = lax.axis_index("s") * NC + lax.axis_index("c")
    lane = lax.iota(jnp.int32, 16)
    zero16 = lane * 0
    rowvs = [lane + 16 * g for g in range(16)]

    def in_copies(c, buf):
        return [
            pltpu.make_async_copy(
                tt_hbm.at[pl.ds(8 * s, 8), pl.ds(c * RW, RW)],
                tt_v.at[buf, pl.ds(8 * s, 8), :],
                isem.at[buf],
            )
            for s in range(NSUB)
        ]

    def in_start(c, buf):
        for cp in in_copies(c, buf):
            cp.start()

    def in_wait(c, buf):
        for cp in in_copies(c, buf):
            cp.wait()

    def out_copy(c, half):
        return pltpu.make_async_copy(
            out_v,
            out_hbm.at[pl.ds(c * (RW // 2) + (RW // 4) * half, RW // 4), :],
            osem,
        )

    def halves(c, buf, skip_first_wait=False):
        for h in range(2):
            _repack_chunk(tt_v, out_p, buf, rowvs, zero16, h)
            if h or not skip_first_wait:
                out_copy(c, h).wait()
            _compact_half(out_p, out_v)
            out_copy(c, h).start()

    in_start(wid, 0)
    in_start(wid + NW, 1)

    in_wait(wid, 0)
    in_start(wid + 2 * NW, 0)
    halves(wid, 0, skip_first_wait=True)

    def step(k, _):
        cA = wid + NW * (2 * k + 1)
        in_wait(cA, 1)

        @pl.when(2 * k + 3 < PER_TILE)
        def _():
            in_start(wid + NW * (2 * k + 3), 1)

        halves(cA, 1)

        cB = wid + NW * (2 * k + 2)
        in_wait(cB, 0)

        @pl.when(2 * k + 4 < PER_TILE)
        def _():
            in_start(wid + NW * (2 * k + 4), 0)

        halves(cB, 0)
        return 0

    lax.fori_loop(0, (PER_TILE - 1) // 2, step, 0, unroll=1)
    out_copy(wid, 0).wait()

    @pl.when(wid == 0)
    def _():
        in_start(NMAIN, 0)
        in_wait(NMAIN, 0)
        for h in range(2):
            _repack_chunk(tt_v, out_p, 0, rowvs, zero16, h)
            _compact_half(out_p, out_v)
            out_copy(NMAIN, h).start()
            out_copy(NMAIN, h).wait()

    @pl.when(wid == 4)
    def _():
        pltpu.make_async_copy(
            tail_hbm,
            out_hbm.at[pl.ds((VOCAB - VTAIL) // 2, VTAIL // 2), :],
            isem.at[0],
        ).start()
        pltpu.make_async_copy(
            tail_hbm,
            out_hbm.at[pl.ds((VOCAB - VTAIL) // 2, VTAIL // 2), :],
            isem.at[0],
        ).wait()


@jax.jit
def _sc_repack(tt, tail):
    mesh = plsc.VectorSubcoreMesh(core_axis_name="c", subcore_axis_name="s")
    return pl.kernel(
        _repack_body,
        out_type=jax.ShapeDtypeStruct((VOCAB // 2, 2 * EMB), jnp.float32),
        mesh=mesh,
        compiler_params=pltpu.CompilerParams(needs_layout_passes=False),
        scratch_types=[
            pltpu.VMEM((2, EMB, RW), jnp.float32),
            pltpu.VMEM((RW // 2, EMB + 1), jnp.float32),
            pltpu.VMEM((RW // 4, 2 * EMB), jnp.float32),
            pltpu.SemaphoreType.DMA((2,)),
            pltpu.SemaphoreType.DMA,
        ],
    )(tt, tail)


def _gather_start(table_hbm, idx_t, rows_v, sem, j, buf):
    pltpu.make_async_copy(
        table_hbm.at[idx_t.at[j, pl.ds(0, CHUNK0)]],
        rows_v.at[buf, pl.ds(0, CHUNK0), :],
        sem.at[buf],
    ).start()
    pltpu.make_async_copy(
        table_hbm.at[idx_t.at[j, pl.ds(CHUNK0, CHUNK1)]],
        rows_v.at[buf, pl.ds(CHUNK0, CHUNK1), :],
        sem.at[buf],
    ).start()


def _gather_wait(table_hbm, idx_t, rows_v, sem, j, buf):
    pltpu.make_async_copy(
        table_hbm.at[idx_t.at[j, pl.ds(0, CHUNK0)]],
        rows_v.at[buf, pl.ds(0, CHUNK0), :],
        sem.at[buf],
    ).wait()
    pltpu.make_async_copy(
        table_hbm.at[idx_t.at[j, pl.ds(CHUNK0, CHUNK1)]],
        rows_v.at[buf, pl.ds(CHUNK0, CHUNK1), :],
        sem.at[buf],
    ).wait()


def _accumulate(rows_v, acc_v, j, buf):
    def body(l, carry):
        a0, a1, a2, a3 = carry
        a0 = a0 + rows_v[buf, l, pl.ds(0, 16)]
        a1 = a1 + rows_v[buf, l, pl.ds(16, 16)]
        a2 = a2 + rows_v[buf, l, pl.ds(32, 16)]
        a3 = a3 + rows_v[buf, l, pl.ds(48, 16)]
        return (a0, a1, a2, a3)

    z = jnp.zeros((16,), jnp.float32)
    a0, a1, a2, a3 = lax.fori_loop(0, SEQ, body, (z, z, z, z), unroll=4)
    acc_v[j, pl.ds(0, 16)] = a0
    acc_v[j, pl.ds(16, 16)] = a1
    acc_v[j, pl.ds(32, 16)] = a2
    acc_v[j, pl.ds(48, 16)] = a3


def _transpose_idx(idx_v, idx_t):
    lane = lax.iota(jnp.int32, 16)

    def body(j, _):
        js = jnp.full((16,), 0, jnp.int32) + j
        for lb in range(SEQ // 16):
            lv = lane + (lb * 16)
            vals = plsc.load_gather(idx_v, [lv, js])
            idx_t[j, pl.ds(lb * 16, 16)] = vals
        lv = lane + (SEQ - 16)
        vals = plsc.load_gather(idx_v, [lv, js])
        idx_t[j, pl.ds(SEQ - 16, 16)] = vals
        return 0

    lax.fori_loop(0, B_PER_W, body, 0, unroll=2)


def _sc_body(x_hbm, table_hbm, out_hbm, idx_v, idx_t, rows_v, acc_v, sem):
    wid = lax.axis_index("s") * NC + lax.axis_index("c")
    base = wid * B_PER_W

    pltpu.sync_copy(x_hbm.at[:, pl.ds(base, B_PER_W)], idx_v)
    _transpose_idx(idx_v, idx_t)

    _gather_start(table_hbm, idx_t, rows_v, sem, 0, 0)
    _gather_start(table_hbm, idx_t, rows_v, sem, 1, 1)

    def step(j0, _):
        _gather_wait(table_hbm, idx_t, rows_v, sem, j0, 0)
        _gather_start(table_hbm, idx_t, rows_v, sem, j0 + 2, 0)
        _accumulate(rows_v, acc_v, j0, 0)
        _gather_wait(table_hbm, idx_t, rows_v, sem, j0 + 1, 1)
        _gather_start(table_hbm, idx_t, rows_v, sem, j0 + 3, 1)
        _accumulate(rows_v, acc_v, j0 + 1, 1)
        return 0

    lax.fori_loop(0, (B_PER_W - 2) // 2, lambda i, c: step(2 * i, c), 0,
                  unroll=1)

    _gather_wait(table_hbm, idx_t, rows_v, sem, B_PER_W - 2, 0)
    _accumulate(rows_v, acc_v, B_PER_W - 2, 0)
    _gather_wait(table_hbm, idx_t, rows_v, sem, B_PER_W - 1, 1)
    _accumulate(rows_v, acc_v, B_PER_W - 1, 1)

    pltpu.sync_copy(acc_v, out_hbm.at[pl.ds(base, B_PER_W), :])


@jax.jit
def _sc_sums(x, table):
    mesh = plsc.VectorSubcoreMesh(core_axis_name="c", subcore_axis_name="s")
    return pl.kernel(
        _sc_body,
        out_type=jax.ShapeDtypeStruct((BATCH, EMB), jnp.float32),
        mesh=mesh,
        compiler_params=pltpu.CompilerParams(use_tc_tiling_on_sc=False,
                                             needs_layout_passes=False),
        scratch_types=[
            pltpu.VMEM((SEQ, B_PER_W), jnp.int32),
            pltpu.VMEM((B_PER_W, SEQ), jnp.int32),
            pltpu.VMEM((2, SEQ, EMB), jnp.float32),
            pltpu.VMEM((B_PER_W, EMB), jnp.float32),
            pltpu.SemaphoreType.DMA((2,)),
        ],
    )(x, table)


def _tc_body(sums_ref, invlen_ref, w_ref, b_ref, out_ref):
    mean = sums_ref[:] * invlen_ref[:]
    out = lax.dot_general(mean, w_ref[:], (((1,), (1,)), ((), ())),
                          preferred_element_type=jnp.float32)
    out_ref[:] = out + b_ref[:]


@jax.jit
def _tc_linear(sums, lengths, W, b):
    invlen = (1.0 / lengths.astype(jnp.float32))[:, None]
    return pl.pallas_call(
        _tc_body,
        out_shape=jax.ShapeDtypeStruct((BATCH, W.shape[0]), jnp.float32),
    )(sums, invlen, W, b[None, :])


def kernel(x, lengths, table, W, b):
    tt = table.T
    tail = table[VOCAB - VTAIL:].reshape(VTAIL // 2, 2 * EMB)
    packed = _sc_repack(tt, tail)
    sums = _sc_sums(x, packed.reshape(VOCAB, EMB))
    return _tc_linear(sums, lengths, W, b)


# speedup vs baseline: 2.5799x; 2.5799x over previous
import functools

import jax
import jax.numpy as jnp
from jax import lax
from jax.experimental import pallas as pl
from jax.experimental.pallas import tpu as pltpu
from jax.experimental.pallas import tpu_sc as plsc

VOCAB = 1000000
EMB = 64
SEQ = 200
BATCH = 4096

NC = 2
NS = 16
NW = NC * NS
B_PER_W = BATCH // NW

CHUNK0 = 128
CHUNK1 = SEQ - CHUNK0


def _gather_start(table_hbm, idx_t, rows_v, sem, j, buf):
    pltpu.make_async_copy(
        table_hbm.at[idx_t.at[j, pl.ds(0, CHUNK0)]],
        rows_v.at[buf, pl.ds(0, CHUNK0), :],
        sem.at[buf],
    ).start()
    pltpu.make_async_copy(
        table_hbm.at[idx_t.at[j, pl.ds(CHUNK0, CHUNK1)]],
        rows_v.at[buf, pl.ds(CHUNK0, CHUNK1), :],
        sem.at[buf],
    ).start()


def _gather_wait(table_hbm, idx_t, rows_v, sem, j, buf):
    pltpu.make_async_copy(
        table_hbm.at[idx_t.at[j, pl.ds(0, CHUNK0)]],
        rows_v.at[buf, pl.ds(0, CHUNK0), :],
        sem.at[buf],
    ).wait()
    pltpu.make_async_copy(
        table_hbm.at[idx_t.at[j, pl.ds(CHUNK0, CHUNK1)]],
        rows_v.at[buf, pl.ds(CHUNK0, CHUNK1), :],
        sem.at[buf],
    ).wait()


def _accumulate(rows_v, acc_v, j, buf):
    def body(l, carry):
        a0, a1, a2, a3 = carry
        a0 = a0 + rows_v[buf, l, pl.ds(0, 16)]
        a1 = a1 + rows_v[buf, l, pl.ds(16, 16)]
        a2 = a2 + rows_v[buf, l, pl.ds(32, 16)]
        a3 = a3 + rows_v[buf, l, pl.ds(48, 16)]
        return (a0, a1, a2, a3)

    z = jnp.zeros((16,), jnp.float32)
    a0, a1, a2, a3 = lax.fori_loop(0, SEQ, body, (z, z, z, z), unroll=4)
    acc_v[j, pl.ds(0, 16)] = a0
    acc_v[j, pl.ds(16, 16)] = a1
    acc_v[j, pl.ds(32, 16)] = a2
    acc_v[j, pl.ds(48, 16)] = a3


def _sc_body(xt_hbm, table_hbm, out_hbm, idx_t, rows_v, acc_v, sem):
    wid = lax.axis_index("s") * NC + lax.axis_index("c")
    base = wid * B_PER_W

    pltpu.sync_copy(xt_hbm.at[pl.ds(base, B_PER_W), :], idx_t)

    _gather_start(table_hbm, idx_t, rows_v, sem, 0, 0)
    _gather_start(table_hbm, idx_t, rows_v, sem, 1, 1)

    def step(j0, _):
        _gather_wait(table_hbm, idx_t, rows_v, sem, j0, 0)
        _gather_start(table_hbm, idx_t, rows_v, sem, j0 + 2, 0)
        _accumulate(rows_v, acc_v, j0, 0)
        _gather_wait(table_hbm, idx_t, rows_v, sem, j0 + 1, 1)
        _gather_start(table_hbm, idx_t, rows_v, sem, j0 + 3, 1)
        _accumulate(rows_v, acc_v, j0 + 1, 1)
        return 0

    lax.fori_loop(0, (B_PER_W - 2) // 2, lambda i, c: step(2 * i, c), 0,
                  unroll=1)

    _gather_wait(table_hbm, idx_t, rows_v, sem, B_PER_W - 2, 0)
    _accumulate(rows_v, acc_v, B_PER_W - 2, 0)
    _gather_wait(table_hbm, idx_t, rows_v, sem, B_PER_W - 1, 1)
    _accumulate(rows_v, acc_v, B_PER_W - 1, 1)

    pltpu.sync_copy(acc_v, out_hbm.at[pl.ds(base, B_PER_W), :])


@jax.jit
def _sc_sums(xt, table):
    mesh = plsc.VectorSubcoreMesh(core_axis_name="c", subcore_axis_name="s")
    return pl.kernel(
        _sc_body,
        out_type=jax.ShapeDtypeStruct((BATCH, EMB), jnp.float32),
        mesh=mesh,
        compiler_params=pltpu.CompilerParams(use_tc_tiling_on_sc=False,
                                             needs_layout_passes=False),
        scratch_types=[
            pltpu.VMEM((B_PER_W, SEQ), jnp.int32),
            pltpu.VMEM((2, SEQ, EMB), jnp.float32),
            pltpu.VMEM((B_PER_W, EMB), jnp.float32),
            pltpu.SemaphoreType.DMA((2,)),
        ],
    )(xt, table)


def _tc_body(sums_ref, invlen_ref, w_ref, b_ref, out_ref):
    mean = sums_ref[:] * invlen_ref[:]
    out = lax.dot_general(mean, w_ref[:], (((1,), (1,)), ((), ())),
                          preferred_element_type=jnp.float32)
    out_ref[:] = out + b_ref[:]


@jax.jit
def _tc_linear(sums, lengths, W, b):
    invlen = (1.0 / lengths.astype(jnp.float32))[:, None]
    return pl.pallas_call(
        _tc_body,
        out_shape=jax.ShapeDtypeStruct((BATCH, W.shape[0]), jnp.float32),
    )(sums, invlen, W, b[None, :])


def kernel(x, lengths, table, W, b):
    sums = _sc_sums(x.T, table)
    return _tc_linear(sums, lengths, W, b)
